# Initial kernel scaffold; baseline (speedup 1.0000x reference)
import jax
import jax.numpy as jnp
from jax.experimental import pallas as pl


def kernel(features, rel_emb, adj, r_index, r_val, proxy, gate_w, gate_b, attn_k0, attn_k1):
    raise NotImplementedError("write your pallas kernel here")


# speedup vs baseline: 10.0889x; 10.0889x over previous
import functools

import jax
import jax.numpy as jnp
from jax import lax
from jax.experimental import pallas as pl
from jax.experimental.pallas import tpu as pltpu
from jax.experimental.pallas import tpu_sc as plsc

NODE = 10000
E = 320000
RELS = 1000
D = 128
NC, NS, L = 2, 16, 16
NW = NC * NS
CB = 128
NCHUNK = E // CB
ROWS_S = NODE // NS

_SC_MESH = plsc.VectorSubcoreMesh(core_axis_name="c", subcore_axis_name="s")


def _prep_body(feats_ref, rel_ref, ak0_ref, ak1_ref, feat0_ref, nrel_ref, a01_ref):
    feat0_ref[...] = jnp.tanh(feats_ref[...])
    r = rel_ref[...]
    n = jnp.sqrt(jnp.sum(r * r, axis=1, keepdims=True))
    nr = r / jnp.maximum(n, 1e-12)
    nrel_ref[...] = nr
    a0 = nr @ ak0_ref[...]
    a1 = nr @ ak1_ref[...]
    a01_ref[...] = jnp.concatenate([a0, a1], axis=1).T


def _prep(features, rel_emb, ak0, ak1):
    return pl.pallas_call(
        _prep_body,
        out_shape=(
            jax.ShapeDtypeStruct((NODE, D), jnp.float32),
            jax.ShapeDtypeStruct((RELS, D), jnp.float32),
            jax.ShapeDtypeStruct((2, RELS), jnp.float32),
        ),
    )(features, rel_emb, ak0, ak1)


def _weights_body(rel_hbm, rval_hbm, seg_hbm, a01_hbm, den0_out, den1_out,
                  a0_v, a1_v, rel_v, rval_v, seg_v, den0_v, den1_v):
    s = lax.axis_index("s")
    c = lax.axis_index("c")
    wid = s * NC + c
    pltpu.sync_copy(a01_hbm.at[0], a0_v)
    pltpu.sync_copy(a01_hbm.at[1], a1_v)

    zero = jnp.zeros((L,), jnp.float32)
    zidx = jnp.zeros((L,), jnp.int32)

    def z(i, carry):
        den0_v[0, pl.ds(i * L, L)] = zero
        den1_v[0, pl.ds(i * L, L)] = zero
        return carry

    lax.fori_loop(0, NODE // L, z, 0)

    kb = 2000

    def chunk(ci, carry):
        off = wid * (E // NW) + ci * kb
        pltpu.sync_copy(rel_hbm.at[pl.ds(off, kb)], rel_v)
        pltpu.sync_copy(rval_hbm.at[pl.ds(off, kb)], rval_v)
        pltpu.sync_copy(seg_hbm.at[pl.ds(off, kb)], seg_v)

        def grp(j, carry2):
            idx = rel_v[pl.ds(j * L, L)]
            sgn = jnp.sign(rval_v[pl.ds(j * L, L)])
            sv = seg_v[pl.ds(j * L, L)]
            g0 = plsc.load_gather(a0_v, [idx])
            plsc.addupdate_scatter(den0_v, [zidx, sv], jnp.exp(sgn * g0))
            g1 = plsc.load_gather(a1_v, [idx])
            plsc.addupdate_scatter(den1_v, [zidx, sv], jnp.exp(sgn * g1))
            return carry2

        lax.fori_loop(0, kb // L, grp, 0)
        return carry

    lax.fori_loop(0, (E // NW) // kb, chunk, 0)
    pltpu.sync_copy(den0_v, den0_out.at[wid])
    pltpu.sync_copy(den1_v, den1_out.at[wid])


def _weights(rrel, r_val, seg, a01):
    kb = 2000
    f = pl.kernel(
        _weights_body,
        out_type=(
            jax.ShapeDtypeStruct((NW, 1, NODE), jnp.float32),
            jax.ShapeDtypeStruct((NW, 1, NODE), jnp.float32),
        ),
        mesh=_SC_MESH,
        compiler_params=pltpu.CompilerParams(needs_layout_passes=False),
        scratch_types=[
            pltpu.VMEM((RELS,), jnp.float32),
            pltpu.VMEM((RELS,), jnp.float32),
            pltpu.VMEM((kb,), jnp.int32),
            pltpu.VMEM((kb,), jnp.float32),
            pltpu.VMEM((kb,), jnp.int32),
            pltpu.VMEM((1, NODE), jnp.float32),
            pltpu.VMEM((1, NODE), jnp.float32),
        ],
    )
    return f(rrel, r_val, seg, a01)


def _recip_body(den0_ref, den1_ref, rden_ref):
    den = jnp.concatenate(
        [jnp.sum(den0_ref[...], axis=0), jnp.sum(den1_ref[...], axis=0)],
        axis=0)
    rden_ref[...] = jnp.where(den > 0.0, 1.0 / den, 0.0)


def _recip(den0_part, den1_part):
    return pl.pallas_call(
        _recip_body,
        out_shape=jax.ShapeDtypeStruct((2, NODE), jnp.float32),
    )(den0_part, den1_part)


def _layer_body(feat_hbm, nrel_hbm, src_hbm, seg_hbm, rel_hbm, rval_hbm,
                a_hbm, rden_hbm, acc_out,
                a_v, rden_v, src_v, seg_v, rel_v, rval_v, w_v, s2_v,
                frows_v, nrows_v, acc_sh, sem_f, sem_n):
    s = lax.axis_index("s")
    c = lax.axis_index("c")
    wid = s * NC + c
    pltpu.sync_copy(a_hbm, a_v)
    pltpu.sync_copy(rden_hbm, rden_v)

    zero = jnp.zeros((L,), jnp.float32)

    def zrow(i, carry):
        def zcol(k, carry2):
            nrows_v[i, pl.ds(k * L, L)] = zero
            return carry2
        return lax.fori_loop(0, D // L, zcol, carry)

    lax.fori_loop(0, CB, zrow, 0)
    for i in range(6):
        pltpu.sync_copy(nrows_v.at[pl.ds(0, 104)],
                        acc_sh.at[pl.ds(s * 624 + i * 104, 104)])

    @pl.when(s == NS - 1)
    def _():
        pltpu.sync_copy(nrows_v.at[pl.ds(0, 16)],
                        acc_sh.at[pl.ds(NS * 624, 16)])

    plsc.subcore_barrier()

    nfull = NCHUNK // NW
    extra = NCHUNK - nfull * NW
    my_n = nfull + jnp.where(wid < extra, 1, 0)

    def chunk(ci, carry):
        off = (wid + ci * NW) * CB
        pltpu.sync_copy(src_hbm.at[pl.ds(off, CB)], src_v)
        pltpu.sync_copy(seg_hbm.at[pl.ds(off, CB)], seg_v)
        pltpu.sync_copy(rel_hbm.at[pl.ds(off, CB)], rel_v)
        pltpu.sync_copy(rval_hbm.at[pl.ds(off, CB)], rval_v)
        cf = pltpu.async_copy(feat_hbm.at[src_v], frows_v, sem_f)
        cn = pltpu.async_copy(nrel_hbm.at[rel_v], nrows_v, sem_n)

        def grp(j, carry2):
            idx = rel_v[pl.ds(j * L, L)]
            sgn = jnp.sign(rval_v[pl.ds(j * L, L)])
            sv = seg_v[pl.ds(j * L, L)]
            g = plsc.load_gather(a_v, [idx])
            rd = plsc.load_gather(rden_v, [sv])
            w_v[pl.ds(j * L, L)] = jnp.exp(sgn * g) * rd
            s2_v[pl.ds(j * L, L)] = sgn * sgn
            return carry2

        lax.fori_loop(0, CB // L, grp, 0)
        cf.wait()
        cn.wait()

        def tri(j, carry3):
            wv = w_v[pl.ds(j * L, L)]
            sv2 = s2_v[pl.ds(j * L, L)]
            for tt in range(L):
                t = j * L + tt
                fr = [frows_v[t, pl.ds(k * L, L)] for k in range(D // L)]
                nr = [nrows_v[t, pl.ds(k * L, L)] for k in range(D // L)]
                acc = fr[0] * nr[0]
                for k in range(1, D // L):
                    acc = acc + fr[k] * nr[k]
                m2 = 2.0 * jnp.sum(acc) * sv2[tt]
                w = wv[tt]
                for k in range(D // L):
                    frows_v[t, pl.ds(k * L, L)] = w * (fr[k] - m2 * nr[k])
            return carry3

        lax.fori_loop(0, CB // L, tri, 0)
        pltpu.sync_copy(frows_v, acc_sh.at[seg_v], add=True)
        return carry

    lax.fori_loop(0, my_n, chunk, 0)
    plsc.subcore_barrier()
    for i in range(6):
        pltpu.sync_copy(acc_sh.at[pl.ds(s * 624 + i * 104, 104)],
                        acc_out.at[c, pl.ds(s * 624 + i * 104, 104)])

    @pl.when(s == NS - 1)
    def _():
        pltpu.sync_copy(acc_sh.at[pl.ds(NS * 624, 16)],
                        acc_out.at[c, pl.ds(NS * 624, 16)])


def _layer(feat, nrel, src, seg, rrel, r_val, a_k, rden_k):
    f = pl.kernel(
        _layer_body,
        out_type=jax.ShapeDtypeStruct((NC, NODE, D), jnp.float32),
        mesh=_SC_MESH,
        compiler_params=pltpu.CompilerParams(needs_layout_passes=False),
        scratch_types=[
            pltpu.VMEM((RELS,), jnp.float32),
            pltpu.VMEM((NODE,), jnp.float32),
            pltpu.VMEM((CB,), jnp.int32),
            pltpu.VMEM((CB,), jnp.int32),
            pltpu.VMEM((CB,), jnp.int32),
            pltpu.VMEM((CB,), jnp.float32),
            pltpu.VMEM((CB,), jnp.float32),
            pltpu.VMEM((CB,), jnp.float32),
            pltpu.VMEM((CB, D), jnp.float32),
            pltpu.VMEM((CB, D), jnp.float32),
            pltpu.VMEM_SHARED((NODE, D), jnp.float32),
            pltpu.SemaphoreType.DMA,
            pltpu.SemaphoreType.DMA,
        ],
    )
    return f(feat, nrel, src, seg, rrel, r_val, a_k, rden_k)


def _combine_body(acc_ref, out_ref):
    out_ref[...] = jnp.tanh(acc_ref[0] + acc_ref[1])


def _combine(acc_part):
    return pl.pallas_call(
        _combine_body,
        out_shape=jax.ShapeDtypeStruct((NODE, D), jnp.float32),
    )(acc_part)


def _epi_body(x0_ref, x1_ref, x2_ref, proxy_ref, gw_ref, gb_ref, out_ref):
    o = jnp.concatenate([x0_ref[...], x1_ref[...], x2_ref[...]], axis=1)
    on = o / jnp.maximum(
        jnp.sqrt(jnp.sum(o * o, axis=1, keepdims=True)), 1e-12)
    p = proxy_ref[...]
    pn = p / jnp.maximum(
        jnp.sqrt(jnp.sum(p * p, axis=1, keepdims=True)), 1e-12)
    logits = on @ pn.T
    m = jnp.max(logits, axis=1, keepdims=True)
    ex = jnp.exp(logits - m)
    att = ex / jnp.sum(ex, axis=1, keepdims=True)
    pf = o - att @ p
    g = jax.nn.sigmoid(pf @ gw_ref[...].T + gb_ref[...])
    out_ref[...] = g * o + (1.0 - g) * pf


def _epilogue(x0, x1, x2, proxy, gate_w, gate_b):
    nb = 1000
    grid = NODE // nb
    return pl.pallas_call(
        _epi_body,
        grid=(grid,),
        in_specs=[
            pl.BlockSpec((nb, D), lambda i: (i, 0)),
            pl.BlockSpec((nb, D), lambda i: (i, 0)),
            pl.BlockSpec((nb, D), lambda i: (i, 0)),
            pl.BlockSpec((64, 3 * D), lambda i: (0, 0)),
            pl.BlockSpec((3 * D, 3 * D), lambda i: (0, 0)),
            pl.BlockSpec((1, 3 * D), lambda i: (0, 0)),
        ],
        out_specs=pl.BlockSpec((nb, 3 * D), lambda i: (i, 0)),
        out_shape=jax.ShapeDtypeStruct((NODE, 3 * D), jnp.float32),
    )(x0, x1, x2, proxy, gate_w, gate_b.reshape(1, 3 * D))


def kernel(features, rel_emb, adj, r_index, r_val, proxy, gate_w, gate_b,
           attn_k0, attn_k1):
    seg = adj[0]
    src = adj[1]
    rrel = r_index[1]
    feat0, nrel, a01 = _prep(features, rel_emb, attn_k0, attn_k1)
    den0_part, den1_part = _weights(rrel, r_val, seg, a01)
    rden = _recip(den0_part, den1_part)
    feats = [feat0]
    f = feat0
    for k in range(2):
        acc_part = _layer(f, nrel, src, seg, rrel, r_val, a01[k], rden[k])
        f = _combine(acc_part)
        feats.append(f)
    return _epilogue(feats[0], feats[1], feats[2], proxy, gate_w, gate_b)
